# Initial kernel scaffold; baseline (speedup 1.0000x reference)
import jax
import jax.numpy as jnp
from jax.experimental import pallas as pl


def kernel(x, embedding_weight):
    raise NotImplementedError("write your pallas kernel here")


# speedup vs baseline: 2.2434x; 2.2434x over previous
import functools

import jax
import jax.numpy as jnp
from jax import lax
from jax.experimental import pallas as pl
from jax.experimental.pallas import tpu as pltpu
from jax.experimental.pallas import tpu_sc as plsc

D = 1024
B = 4 * 8192
NC, NS = 2, 16
NW = NC * NS
BPW = B // NW
C = 32
NCH = BPW // C


def _gather(table, idx):
  mesh = plsc.VectorSubcoreMesh(core_axis_name="c", subcore_axis_name="s")

  @functools.partial(
      pl.kernel,
      out_type=jax.ShapeDtypeStruct((B, D), jnp.float32),
      mesh=mesh,
      scratch_types=[
          pltpu.VMEM((BPW,), jnp.int32),
          pltpu.VMEM((C, D), jnp.float32),
          pltpu.VMEM((C, D), jnp.float32),
          pltpu.SemaphoreType.DMA,
          pltpu.SemaphoreType.DMA,
          pltpu.SemaphoreType.DMA,
          pltpu.SemaphoreType.DMA,
      ],
  )
  def k(table_hbm, idx_hbm, out_hbm, idx_v, rows0, rows1, si0, si1, so0, so1):
    wid = lax.axis_index("s") * NC + lax.axis_index("c")
    base = wid * BPW
    pltpu.sync_copy(idx_hbm.at[pl.ds(base, BPW)], idx_v)

    def gather(j, rows, sem):
      pltpu.async_copy(table_hbm.at[idx_v.at[pl.ds(j * C, C)]], rows, sem)

    def put(j, rows, sem):
      pltpu.async_copy(rows, out_hbm.at[pl.ds(base + j * C, C)], sem)

    def wait(rows, sem):
      pltpu.make_async_copy(table_hbm.at[pl.ds(0, C)], rows, sem).wait()

    gather(0, rows0, si0)
    gather(1, rows1, si1)

    @pl.loop(0, NCH // 2 - 1)
    def _(i):
      j = i * 2
      wait(rows0, si0)
      put(j, rows0, so0)
      wait(rows1, si1)
      put(j + 1, rows1, so1)
      wait(rows0, so0)
      gather(j + 2, rows0, si0)
      wait(rows1, so1)
      gather(j + 3, rows1, si1)

    wait(rows0, si0)
    pltpu.sync_copy(rows0, out_hbm.at[pl.ds(base + (NCH - 2) * C, C)])
    wait(rows1, si1)
    pltpu.sync_copy(rows1, out_hbm.at[pl.ds(base + (NCH - 1) * C, C)])

  return k(table, idx)


@jax.jit
def kernel(x, embedding_weight):
  idx = x.reshape(-1).astype(jnp.int32)
  out = _gather(embedding_weight, idx)
  return out.reshape(x.shape + (D,))


# speedup vs baseline: 2.2580x; 1.0065x over previous
import functools

import jax
import jax.numpy as jnp
from jax import lax
from jax.experimental import pallas as pl
from jax.experimental.pallas import tpu as pltpu
from jax.experimental.pallas import tpu_sc as plsc

D = 1024
B = 4 * 8192
NC, NS = 2, 16
NW = NC * NS
BPW = B // NW
C = 32
NBUF = 3
NCH = BPW // C
NFULL = (NCH // NBUF) * NBUF


def _gather(table, idx):
  mesh = plsc.VectorSubcoreMesh(core_axis_name="c", subcore_axis_name="s")

  @functools.partial(
      pl.kernel,
      out_type=jax.ShapeDtypeStruct((B, D), jnp.float32),
      mesh=mesh,
      scratch_types=[
          pltpu.VMEM((BPW,), jnp.int32),
          [pltpu.VMEM((C, D), jnp.float32) for _ in range(NBUF)],
          [pltpu.SemaphoreType.DMA for _ in range(NBUF)],
          [pltpu.SemaphoreType.DMA for _ in range(NBUF)],
      ],
  )
  def k(table_hbm, idx_hbm, out_hbm, idx_v, rows, si, so):
    wid = lax.axis_index("s") * NC + lax.axis_index("c")
    base = wid * BPW
    pltpu.sync_copy(idx_hbm.at[pl.ds(base, BPW)], idx_v)

    def gather(j, b):
      pltpu.async_copy(table_hbm.at[idx_v.at[pl.ds(j * C, C)]], rows[b], si[b])

    def put(j, b):
      pltpu.async_copy(rows[b], out_hbm.at[pl.ds(base + j * C, C)], so[b])

    def wait(b, sem):
      pltpu.make_async_copy(table_hbm.at[pl.ds(0, C)], rows[b], sem[b]).wait()

    for b in range(NBUF):
      gather(b, b)

    @pl.loop(0, NFULL // NBUF - 1)
    def _(i):
      j = i * NBUF
      for b in range(NBUF):
        wait(b, si)
        put(j + b, b)
      for b in range(NBUF):
        wait(b, so)
        gather(j + NBUF + b, b)

    jlast = NFULL - NBUF
    for b in range(NBUF):
      wait(b, si)
      put(jlast + b, b)
    for r, j in enumerate(range(NFULL, NCH)):
      b = r % NBUF
      wait(b, so)
      gather(j, b)
      wait(b, si)
      put(j, b)
    for b in range(NBUF):
      wait(b, so)

  return k(table, idx)


@jax.jit
def kernel(x, embedding_weight):
  idx = x.reshape(-1).astype(jnp.int32)
  out = _gather(embedding_weight, idx)
  return out.reshape(x.shape + (D,))


# speedup vs baseline: 2.3406x; 1.0366x over previous
import functools

import jax
import jax.numpy as jnp
from jax import lax
from jax.experimental import pallas as pl
from jax.experimental.pallas import tpu as pltpu
from jax.experimental.pallas import tpu_sc as plsc

D = 1024
B = 4 * 8192
NC, NS = 2, 16
NW = NC * NS
BPW = B // NW
C = 56
NBUF = 2
NFCH = BPW // C
RING = NFCH // NBUF
TAIL = NFCH - RING * NBUF
REM = BPW - NFCH * C


def _gather(table, idx):
  mesh = plsc.VectorSubcoreMesh(core_axis_name="c", subcore_axis_name="s")

  @functools.partial(
      pl.kernel,
      out_type=jax.ShapeDtypeStruct((B, D), jnp.float32),
      mesh=mesh,
      scratch_types=[
          pltpu.VMEM((BPW,), jnp.int32),
          [pltpu.VMEM((C, D), jnp.float32) for _ in range(NBUF)],
          [pltpu.SemaphoreType.DMA for _ in range(NBUF)],
          [pltpu.SemaphoreType.DMA for _ in range(NBUF)],
      ],
  )
  def k(table_hbm, idx_hbm, out_hbm, idx_v, rows, si, so):
    wid = lax.axis_index("s") * NC + lax.axis_index("c")
    base = wid * BPW
    pltpu.sync_copy(idx_hbm.at[pl.ds(base, BPW)], idx_v)

    def gather(j, b, n=C):
      pltpu.async_copy(table_hbm.at[idx_v.at[pl.ds(j * C, n)]],
                       rows[b].at[pl.ds(0, n)], si[b])

    def put(j, b, n=C):
      pltpu.async_copy(rows[b].at[pl.ds(0, n)],
                       out_hbm.at[pl.ds(base + j * C, n)], so[b])

    def wait(b, sem, n=C):
      pltpu.make_async_copy(table_hbm.at[pl.ds(0, n)],
                            rows[b].at[pl.ds(0, n)], sem[b]).wait()

    for b in range(NBUF):
      gather(b, b)

    @pl.loop(0, RING - 1)
    def _(i):
      j = i * NBUF
      for b in range(NBUF):
        wait(b, si)
        put(j + b, b)
      for b in range(NBUF):
        wait(b, so)
        gather(j + NBUF + b, b)

    jlast = (RING - 1) * NBUF
    for b in range(NBUF):
      wait(b, si)
      put(jlast + b, b)

    nxt = 0
    for t in range(TAIL):
      b = nxt % NBUF
      wait(b, so)
      gather(RING * NBUF + t, b)
      wait(b, si)
      put(RING * NBUF + t, b)
      nxt += 1
    if REM:
      b = nxt % NBUF
      wait(b, so)
      gather(NFCH, b, REM)
      wait(b, si, REM)
      put(NFCH, b, REM)

    for b in range(NBUF):
      if REM and b == nxt % NBUF:
        wait(b, so, REM)
      else:
        wait(b, so)

  return k(table, idx)


@jax.jit
def kernel(x, embedding_weight):
  idx = x.reshape(-1).astype(jnp.int32)
  out = _gather(embedding_weight, idx)
  return out.reshape(x.shape + (D,))
